# Initial kernel scaffold; baseline (speedup 1.0000x reference)
import jax
import jax.numpy as jnp
from jax.experimental import pallas as pl


def kernel(nodes, edges, receivers, senders, W_node, W_edge):
    raise NotImplementedError("write your pallas kernel here")


# speedup vs baseline: 2.4932x; 2.4932x over previous
import functools

import jax
import jax.numpy as jnp
from jax import lax
from jax.experimental import pallas as pl
from jax.experimental.pallas import tpu as pltpu
from jax.experimental.pallas import tpu_sc as plsc

N_NODES = 10000
N_EDGES = 320000
D = 128

NC = 2
NS = 16
NW = NC * NS


def _transpose_body(x_ref, o_ref):
    o_ref[...] = x_ref[...].T


def _transpose(x, bn):
    d, m = x.shape
    return pl.pallas_call(
        _transpose_body,
        grid=(m // bn,),
        in_specs=[pl.BlockSpec((d, bn), lambda i: (0, i))],
        out_specs=pl.BlockSpec((bn, d), lambda i: (i, 0)),
        out_shape=jax.ShapeDtypeStruct((m, d), x.dtype),
    )(x)


def _node_update_body(last, n_ref, s_ref, r_ref, wn_ref, we_ref,
                      no_ref, a_ref, b_ref):
    wn = wn_ref[...]
    x = jnp.dot(n_ref[...], wn[:D], preferred_element_type=jnp.float32)
    x = x + jnp.dot(s_ref[...], wn[D:2 * D], preferred_element_type=jnp.float32)
    x = x + jnp.dot(r_ref[...], wn[2 * D:], preferred_element_type=jnp.float32)
    if last:
        no_ref[...] = x.T
    else:
        no_ref[...] = x
    we = we_ref[...]
    a_ref[...] = jnp.dot(x, we[D:2 * D], preferred_element_type=jnp.float32)
    b_ref[...] = jnp.dot(x, we[2 * D:], preferred_element_type=jnp.float32)


def _node_update(nodes_t, s_t, r_t, wn_t, we_t, last):
    bn = N_NODES
    grid = (N_NODES // bn,)
    row_spec = pl.BlockSpec((bn, D), lambda i: (i, 0))
    w_spec = pl.BlockSpec((3 * D, D), lambda i: (0, 0))
    if last:
        no_spec = pl.BlockSpec((D, bn), lambda i: (0, i))
        no_shape = jax.ShapeDtypeStruct((D, N_NODES), jnp.float32)
    else:
        no_spec = row_spec
        no_shape = jax.ShapeDtypeStruct((N_NODES, D), jnp.float32)
    return pl.pallas_call(
        functools.partial(_node_update_body, last),
        grid=grid,
        in_specs=[row_spec, row_spec, row_spec, w_spec, w_spec],
        out_specs=(no_spec, row_spec, row_spec),
        out_shape=(no_shape,
                   jax.ShapeDtypeStruct((N_NODES, D), jnp.float32),
                   jax.ShapeDtypeStruct((N_NODES, D), jnp.float32)),
    )(nodes_t, s_t, r_t, wn_t, we_t)


def _edge_update_body(last, e_ref, ga_ref, gb_ref, w_ref, o_ref):
    y = jnp.dot(e_ref[...], w_ref[...], preferred_element_type=jnp.float32)
    y = y + ga_ref[...] + gb_ref[...]
    if last:
        o_ref[...] = y.T
    else:
        o_ref[...] = y


def _edge_update(edges_t, ga, gb, we0_t, last):
    be = 2560
    grid = (N_EDGES // be,)
    row_spec = pl.BlockSpec((be, D), lambda i: (i, 0))
    w_spec = pl.BlockSpec((D, D), lambda i: (0, 0))
    if last:
        o_spec = pl.BlockSpec((D, be), lambda i: (0, i))
        o_shape = jax.ShapeDtypeStruct((D, N_EDGES), jnp.float32)
    else:
        o_spec = row_spec
        o_shape = jax.ShapeDtypeStruct((N_EDGES, D), jnp.float32)
    return pl.pallas_call(
        functools.partial(_edge_update_body, last),
        grid=grid,
        in_specs=[row_spec, row_spec, row_spec, w_spec],
        out_specs=o_spec,
        out_shape=o_shape,
    )(edges_t, ga, gb, we0_t)


_SEG_CB = 80
_SEG_EPT = N_EDGES // NS
_SEG_ZCH = 400
_SEG_NCH = N_NODES // _SEG_ZCH

_MESH = plsc.VectorSubcoreMesh(
    core_axis_name="c", subcore_axis_name="s", num_cores=NC)


def _segsum_body(e_ref, s_ref, r_ref, z_ref, so_ref, ro_ref,
                 idx_v, rows_v, acc_sh):
    c = lax.axis_index("c")
    t = lax.axis_index("s")

    for k in range(2):
        ci = t + k * NS

        @pl.when(ci < _SEG_NCH)
        def _():
            sl = pl.ds(ci * _SEG_ZCH, _SEG_ZCH)
            pltpu.sync_copy(z_ref.at[sl], acc_sh.at[sl])

    plsc.subcore_barrier()

    def chunk_loop(idx_hbm):
        def body(i, _):
            base = t * _SEG_EPT + i * _SEG_CB
            pltpu.sync_copy(idx_hbm.at[pl.ds(base, _SEG_CB)], idx_v)
            pltpu.sync_copy(e_ref.at[pl.ds(base, _SEG_CB)], rows_v)
            pltpu.sync_copy(rows_v, acc_sh.at[idx_v], add=True)
            return 0
        lax.fori_loop(0, _SEG_EPT // _SEG_CB, body, 0)

    @pl.when(c == 0)
    def _():
        chunk_loop(s_ref)

    @pl.when(c == 1)
    def _():
        chunk_loop(r_ref)

    plsc.subcore_barrier()

    for k in range(2):
        ci = t + k * NS

        @pl.when(ci < _SEG_NCH)
        def _():
            sl = pl.ds(ci * _SEG_ZCH, _SEG_ZCH)

            @pl.when(c == 0)
            def _():
                pltpu.sync_copy(acc_sh.at[sl], so_ref.at[sl])

            @pl.when(c == 1)
            def _():
                pltpu.sync_copy(acc_sh.at[sl], ro_ref.at[sl])


def _segsum(edges_t, senders, receivers, zeros_nd):
    f = pl.kernel(
        _segsum_body,
        out_type=(jax.ShapeDtypeStruct((N_NODES, D), jnp.float32),
                  jax.ShapeDtypeStruct((N_NODES, D), jnp.float32)),
        mesh=_MESH,
        scratch_types=[
            pltpu.VMEM((_SEG_CB,), jnp.int32),
            pltpu.VMEM((_SEG_CB, D), jnp.float32),
            pltpu.VMEM_SHARED((N_NODES, D), jnp.float32),
        ],
    )
    return f(edges_t, senders, receivers, zeros_nd)


_G_CB = 80
_G_EPW = N_EDGES // NW


def _gather_body(a_ref, b_ref, s_ref, r_ref, ga_ref, gb_ref,
                 sidx_v, ridx_v, bufa_v, bufb_v, sema, semb):
    c = lax.axis_index("c")
    t = lax.axis_index("s")
    wid = t * NC + c

    def body(i, _):
        base = wid * _G_EPW + i * _G_CB
        sl = pl.ds(base, _G_CB)
        pltpu.sync_copy(s_ref.at[sl], sidx_v)
        pltpu.sync_copy(r_ref.at[sl], ridx_v)
        cpa = pltpu.async_copy(a_ref.at[sidx_v], bufa_v, sema)
        cpb = pltpu.async_copy(b_ref.at[ridx_v], bufb_v, semb)
        cpa.wait()
        cpb.wait()
        pltpu.sync_copy(bufa_v, ga_ref.at[sl])
        pltpu.sync_copy(bufb_v, gb_ref.at[sl])
        return 0

    lax.fori_loop(0, _G_EPW // _G_CB, body, 0)


def _gather(a_t, b_t, senders, receivers):
    f = pl.kernel(
        _gather_body,
        out_type=(jax.ShapeDtypeStruct((N_EDGES, D), jnp.float32),
                  jax.ShapeDtypeStruct((N_EDGES, D), jnp.float32)),
        mesh=_MESH,
        scratch_types=[
            pltpu.VMEM((_G_CB,), jnp.int32),
            pltpu.VMEM((_G_CB,), jnp.int32),
            pltpu.VMEM((_G_CB, D), jnp.float32),
            pltpu.VMEM((_G_CB, D), jnp.float32),
            pltpu.SemaphoreType.DMA,
            pltpu.SemaphoreType.DMA,
        ],
    )
    return f(a_t, b_t, senders, receivers)


def kernel(nodes, edges, receivers, senders, W_node, W_edge):
    wn_t = W_node.T
    we_t = W_edge.T
    we0_t = we_t[:D]
    zeros_nd = jnp.zeros((N_NODES, D), jnp.float32)

    edges_t = _transpose(edges, 2560)
    nodes_t = _transpose(nodes, N_NODES)

    nodes_out = None
    edges_out = None
    for rnd in range(2):
        last = rnd == 1
        s_t, r_t = _segsum(edges_t, senders, receivers, zeros_nd)
        n_out, a_t, b_t = _node_update(nodes_t, s_t, r_t, wn_t, we_t, last)
        ga, gb = _gather(a_t, b_t, senders, receivers)
        e_out = _edge_update(edges_t, ga, gb, we0_t, last)
        if last:
            nodes_out, edges_out = n_out, e_out
        else:
            nodes_t, edges_t = n_out, e_out

    return nodes_out, edges_out, receivers, senders


# speedup vs baseline: 3.6390x; 1.4596x over previous
import functools

import jax
import jax.numpy as jnp
from jax import lax
from jax.experimental import pallas as pl
from jax.experimental.pallas import tpu as pltpu
from jax.experimental.pallas import tpu_sc as plsc

N_NODES = 10000
N_EDGES = 320000
D = 128

NC = 2
NS = 16
NW = NC * NS


def _transpose_body(x_ref, o_ref):
    o_ref[...] = x_ref[...].T


def _transpose(x, bn):
    d, m = x.shape
    return pl.pallas_call(
        _transpose_body,
        grid=(m // bn,),
        in_specs=[pl.BlockSpec((d, bn), lambda i: (0, i))],
        out_specs=pl.BlockSpec((bn, d), lambda i: (i, 0)),
        out_shape=jax.ShapeDtypeStruct((m, d), x.dtype),
    )(x)


def _node_update_body(last, n_ref, s_ref, r_ref, wn_ref, we_ref,
                      no_ref, a_ref, b_ref):
    wn = wn_ref[...]
    x = jnp.dot(n_ref[...], wn[:D], preferred_element_type=jnp.float32)
    x = x + jnp.dot(s_ref[...], wn[D:2 * D], preferred_element_type=jnp.float32)
    x = x + jnp.dot(r_ref[...], wn[2 * D:], preferred_element_type=jnp.float32)
    if last:
        no_ref[...] = x.T
    else:
        no_ref[...] = x
    we = we_ref[...]
    a_ref[...] = jnp.dot(x, we[D:2 * D], preferred_element_type=jnp.float32)
    b_ref[...] = jnp.dot(x, we[2 * D:], preferred_element_type=jnp.float32)


def _node_update(nodes_t, s_t, r_t, wn_t, we_t, last):
    bn = N_NODES
    grid = (N_NODES // bn,)
    row_spec = pl.BlockSpec((bn, D), lambda i: (i, 0))
    w_spec = pl.BlockSpec((3 * D, D), lambda i: (0, 0))
    if last:
        no_spec = pl.BlockSpec((D, bn), lambda i: (0, i))
        no_shape = jax.ShapeDtypeStruct((D, N_NODES), jnp.float32)
    else:
        no_spec = row_spec
        no_shape = jax.ShapeDtypeStruct((N_NODES, D), jnp.float32)
    return pl.pallas_call(
        functools.partial(_node_update_body, last),
        grid=grid,
        in_specs=[row_spec, row_spec, row_spec, w_spec, w_spec],
        out_specs=(no_spec, row_spec, row_spec),
        out_shape=(no_shape,
                   jax.ShapeDtypeStruct((N_NODES, D), jnp.float32),
                   jax.ShapeDtypeStruct((N_NODES, D), jnp.float32)),
    )(nodes_t, s_t, r_t, wn_t, we_t)


def _edge_update_body(last, e_ref, ga_ref, gb_ref, w_ref, o_ref):
    y = jnp.dot(e_ref[...], w_ref[...], preferred_element_type=jnp.float32)
    y = y + ga_ref[...] + gb_ref[...]
    if last:
        o_ref[...] = y.T
    else:
        o_ref[...] = y


def _edge_update(edges_t, ga, gb, we0_t, last):
    be = 2560
    grid = (N_EDGES // be,)
    row_spec = pl.BlockSpec((be, D), lambda i: (i, 0))
    w_spec = pl.BlockSpec((D, D), lambda i: (0, 0))
    if last:
        o_spec = pl.BlockSpec((D, be), lambda i: (0, i))
        o_shape = jax.ShapeDtypeStruct((D, N_EDGES), jnp.float32)
    else:
        o_spec = row_spec
        o_shape = jax.ShapeDtypeStruct((N_EDGES, D), jnp.float32)
    return pl.pallas_call(
        functools.partial(_edge_update_body, last),
        grid=grid,
        in_specs=[row_spec, row_spec, row_spec, w_spec],
        out_specs=o_spec,
        out_shape=o_shape,
    )(edges_t, ga, gb, we0_t)


_SEG_CB = 80
_SEG_EPT = N_EDGES // NS
_SEG_NST = _SEG_EPT // _SEG_CB
_SEG_IPH = _SEG_NST // 2
_SEG_ZCH = 400
_SEG_NCH = N_NODES // _SEG_ZCH

_MESH = plsc.VectorSubcoreMesh(
    core_axis_name="c", subcore_axis_name="s", num_cores=NC)


def _segsum_body(e_ref, s2_ref, r2_ref, z_ref, so_ref, ro_ref,
                 idx_all, rows0, rows1, acc_sh, lsem0, lsem1, ssem):
    c = lax.axis_index("c")
    t = lax.axis_index("s")

    for k in range(2):
        ci = t + k * NS

        @pl.when(ci < _SEG_NCH)
        def _():
            sl = pl.ds(ci * _SEG_ZCH, _SEG_ZCH)
            pltpu.sync_copy(z_ref.at[sl], acc_sh.at[sl])

    plsc.subcore_barrier()

    def chunk_loop(idx3_hbm):
        def load_idx(phase):
            pltpu.sync_copy(idx3_hbm.at[2 * t + phase], idx_all)

        def load(k, buf, lsem):
            base = t * _SEG_EPT + k * _SEG_CB
            return pltpu.async_copy(e_ref.at[pl.ds(base, _SEG_CB)], buf, lsem)

        def half(k, buf, lsem, obuf, olsem):
            base = t * _SEG_EPT + k * _SEG_CB
            pltpu.make_async_copy(
                e_ref.at[pl.ds(base, _SEG_CB)], buf, lsem).wait()

            @pl.when(k == _SEG_IPH)
            def _():
                load_idx(1)

            d = pltpu.async_copy(
                buf, acc_sh.at[idx_all.at[lax.rem(k, _SEG_IPH)]],
                ssem, add=True)

            @pl.when(k + 1 < _SEG_NST)
            def _():
                load(k + 1, obuf, olsem)

            d.wait()

        load_idx(0)
        load(0, rows0, lsem0)

        def dstep(i, _):
            half(2 * i, rows0, lsem0, rows1, lsem1)
            half(2 * i + 1, rows1, lsem1, rows0, lsem0)
            return 0

        lax.fori_loop(0, _SEG_NST // 2, dstep, 0)

    @pl.when(c == 0)
    def _():
        chunk_loop(s2_ref)

    @pl.when(c == 1)
    def _():
        chunk_loop(r2_ref)

    plsc.subcore_barrier()

    for k in range(2):
        ci = t + k * NS

        @pl.when(ci < _SEG_NCH)
        def _():
            sl = pl.ds(ci * _SEG_ZCH, _SEG_ZCH)

            @pl.when(c == 0)
            def _():
                pltpu.sync_copy(acc_sh.at[sl], so_ref.at[sl])

            @pl.when(c == 1)
            def _():
                pltpu.sync_copy(acc_sh.at[sl], ro_ref.at[sl])


def _segsum(edges_t, senders2d, receivers2d, zeros_nd):
    f = pl.kernel(
        _segsum_body,
        out_type=(jax.ShapeDtypeStruct((N_NODES, D), jnp.float32),
                  jax.ShapeDtypeStruct((N_NODES, D), jnp.float32)),
        mesh=_MESH,
        scratch_types=[
            pltpu.VMEM((_SEG_IPH, _SEG_CB), jnp.int32),
            pltpu.VMEM((_SEG_CB, D), jnp.float32),
            pltpu.VMEM((_SEG_CB, D), jnp.float32),
            pltpu.VMEM_SHARED((N_NODES, D), jnp.float32),
            pltpu.SemaphoreType.DMA,
            pltpu.SemaphoreType.DMA,
            pltpu.SemaphoreType.DMA,
        ],
    )
    return f(edges_t, senders2d, receivers2d, zeros_nd)


_G_CH = 200
_G_GB = 40
_G_EPW = N_EDGES // NW
_G_NST = _G_EPW // _G_CH


def _gather_body(a_ref, b_ref, s_ref, r_ref, ga_ref, gb_ref,
                 sidx_v, ridx_v, bufa0, bufb0, bufa1, bufb1,
                 gsem0, gsem1, wsem0, wsem1):
    c = lax.axis_index("c")
    t = lax.axis_index("s")
    wid = t * NC + c
    tbase = wid * _G_EPW

    pltpu.sync_copy(s_ref.at[pl.ds(tbase, _G_EPW)], sidx_v)
    pltpu.sync_copy(r_ref.at[pl.ds(tbase, _G_EPW)], ridx_v)

    nb = _G_CH // _G_GB

    def g_descs(k, bufa, bufb, gsem):
        ds = []
        for j in range(nb):
            off = k * _G_CH + j * _G_GB
            bsl = pl.ds(j * _G_GB, _G_GB)
            ds.append(pltpu.make_async_copy(
                a_ref.at[sidx_v.at[pl.ds(off, _G_GB)]], bufa.at[bsl], gsem))
            ds.append(pltpu.make_async_copy(
                b_ref.at[ridx_v.at[pl.ds(off, _G_GB)]], bufb.at[bsl], gsem))
        return ds

    def w_descs(k, bufa, bufb, wsem):
        sl = pl.ds(tbase + k * _G_CH, _G_CH)
        return [pltpu.make_async_copy(bufa, ga_ref.at[sl], wsem),
                pltpu.make_async_copy(bufb, gb_ref.at[sl], wsem)]

    bufs = ((bufa0, bufb0, gsem0, wsem0), (bufa1, bufb1, gsem1, wsem1))

    def step(k, cur, oth):
        ca, cb, cg, cw = cur
        oa, ob, og, ow = oth
        for d in g_descs(k, ca, cb, cg):
            d.wait()
        for d in w_descs(k, ca, cb, cw):
            d.start()

        @pl.when(k > 0)
        def _():
            for d in w_descs(k - 1, oa, ob, ow):
                d.wait()

        @pl.when(k + 1 < _G_NST)
        def _():
            for d in g_descs(k + 1, oa, ob, og):
                d.start()

    for d in g_descs(0, bufa0, bufb0, gsem0):
        d.start()

    def dstep(i, _):
        step(2 * i, bufs[0], bufs[1])
        step(2 * i + 1, bufs[1], bufs[0])
        return 0

    lax.fori_loop(0, _G_NST // 2, dstep, 0)

    for d in w_descs(_G_NST - 1, bufa1, bufb1, wsem1):
        d.wait()


def _gather(a_t, b_t, senders, receivers):
    f = pl.kernel(
        _gather_body,
        out_type=(jax.ShapeDtypeStruct((N_EDGES, D), jnp.float32),
                  jax.ShapeDtypeStruct((N_EDGES, D), jnp.float32)),
        mesh=_MESH,
        scratch_types=[
            pltpu.VMEM((_G_EPW,), jnp.int32),
            pltpu.VMEM((_G_EPW,), jnp.int32),
            pltpu.VMEM((_G_CH, D), jnp.float32),
            pltpu.VMEM((_G_CH, D), jnp.float32),
            pltpu.VMEM((_G_CH, D), jnp.float32),
            pltpu.VMEM((_G_CH, D), jnp.float32),
            pltpu.SemaphoreType.DMA,
            pltpu.SemaphoreType.DMA,
            pltpu.SemaphoreType.DMA,
            pltpu.SemaphoreType.DMA,
        ],
    )
    return f(a_t, b_t, senders, receivers)


def kernel(nodes, edges, receivers, senders, W_node, W_edge):
    wn_t = W_node.T
    we_t = W_edge.T
    we0_t = we_t[:D]
    zeros_nd = jnp.zeros((N_NODES, D), jnp.float32)
    senders2d = senders.reshape(NS * 2, _SEG_IPH, _SEG_CB)
    receivers2d = receivers.reshape(NS * 2, _SEG_IPH, _SEG_CB)

    edges_t = _transpose(edges, 2560)
    nodes_t = _transpose(nodes, N_NODES)

    nodes_out = None
    edges_out = None
    for rnd in range(2):
        last = rnd == 1
        s_t, r_t = _segsum(edges_t, senders2d, receivers2d, zeros_nd)
        n_out, a_t, b_t = _node_update(nodes_t, s_t, r_t, wn_t, we_t, last)
        ga, gb = _gather(a_t, b_t, senders, receivers)
        e_out = _edge_update(edges_t, ga, gb, we0_t, last)
        if last:
            nodes_out, edges_out = n_out, e_out
        else:
            nodes_t, edges_t = n_out, e_out

    return nodes_out, edges_out, receivers, senders


# speedup vs baseline: 3.6414x; 1.0006x over previous
import functools

import jax
import jax.numpy as jnp
from jax import lax
from jax.experimental import pallas as pl
from jax.experimental.pallas import tpu as pltpu
from jax.experimental.pallas import tpu_sc as plsc

N_NODES = 10000
N_EDGES = 320000
D = 128

NC = 2
NS = 16
NW = NC * NS


def _transpose_body(x_ref, o_ref):
    o_ref[...] = x_ref[...].T


def _transpose(x, bn):
    d, m = x.shape
    return pl.pallas_call(
        _transpose_body,
        grid=(m // bn,),
        in_specs=[pl.BlockSpec((d, bn), lambda i: (0, i))],
        out_specs=pl.BlockSpec((bn, d), lambda i: (i, 0)),
        out_shape=jax.ShapeDtypeStruct((m, d), x.dtype),
    )(x)


def _node_update_body(last, n_ref, s_ref, r_ref, wn_ref, we_ref,
                      no_ref, a_ref, b_ref):
    wn = wn_ref[...]
    x = jnp.dot(n_ref[...], wn[:D], preferred_element_type=jnp.float32)
    x = x + jnp.dot(s_ref[...], wn[D:2 * D], preferred_element_type=jnp.float32)
    x = x + jnp.dot(r_ref[...], wn[2 * D:], preferred_element_type=jnp.float32)
    if last:
        no_ref[...] = x.T
    else:
        no_ref[...] = x
    we = we_ref[...]
    a_ref[...] = jnp.dot(x, we[D:2 * D], preferred_element_type=jnp.float32)
    b_ref[...] = jnp.dot(x, we[2 * D:], preferred_element_type=jnp.float32)


def _node_update(nodes_t, s_t, r_t, wn_t, we_t, last):
    bn = N_NODES
    grid = (N_NODES // bn,)
    row_spec = pl.BlockSpec((bn, D), lambda i: (i, 0))
    w_spec = pl.BlockSpec((3 * D, D), lambda i: (0, 0))
    if last:
        no_spec = pl.BlockSpec((D, bn), lambda i: (0, i))
        no_shape = jax.ShapeDtypeStruct((D, N_NODES), jnp.float32)
    else:
        no_spec = row_spec
        no_shape = jax.ShapeDtypeStruct((N_NODES, D), jnp.float32)
    return pl.pallas_call(
        functools.partial(_node_update_body, last),
        grid=grid,
        in_specs=[row_spec, row_spec, row_spec, w_spec, w_spec],
        out_specs=(no_spec, row_spec, row_spec),
        out_shape=(no_shape,
                   jax.ShapeDtypeStruct((N_NODES, D), jnp.float32),
                   jax.ShapeDtypeStruct((N_NODES, D), jnp.float32)),
    )(nodes_t, s_t, r_t, wn_t, we_t)


def _edge_update_body(last, e_ref, ga_ref, gb_ref, w_ref, o_ref):
    y = jnp.dot(e_ref[...], w_ref[...], preferred_element_type=jnp.float32)
    y = y + ga_ref[...] + gb_ref[...]
    if last:
        o_ref[...] = y.T
    else:
        o_ref[...] = y


def _edge_update(edges_t, ga, gb, we0_t, last):
    be = 2560
    grid = (N_EDGES // be,)
    row_spec = pl.BlockSpec((be, D), lambda i: (i, 0))
    w_spec = pl.BlockSpec((D, D), lambda i: (0, 0))
    if last:
        o_spec = pl.BlockSpec((D, be), lambda i: (0, i))
        o_shape = jax.ShapeDtypeStruct((D, N_EDGES), jnp.float32)
    else:
        o_spec = row_spec
        o_shape = jax.ShapeDtypeStruct((N_EDGES, D), jnp.float32)
    return pl.pallas_call(
        functools.partial(_edge_update_body, last),
        grid=grid,
        in_specs=[row_spec, row_spec, row_spec, w_spec],
        out_specs=o_spec,
        out_shape=o_shape,
    )(edges_t, ga, gb, we0_t)


_SEG_CB = 80
_SEG_EPT = N_EDGES // NS
_SEG_NST = _SEG_EPT // _SEG_CB
_SEG_IPH = _SEG_NST // 2
_SEG_ZCH = 400
_SEG_NCH = N_NODES // _SEG_ZCH

_MESH = plsc.VectorSubcoreMesh(
    core_axis_name="c", subcore_axis_name="s", num_cores=NC)


def _segsum_body(e_ref, s2_ref, r2_ref, z_ref, so_ref, ro_ref,
                 idx_all, rows0, rows1, acc_sh, lsem0, lsem1, ssem0, ssem1):
    c = lax.axis_index("c")
    t = lax.axis_index("s")

    for k in range(2):
        ci = t + k * NS

        @pl.when(ci < _SEG_NCH)
        def _():
            sl = pl.ds(ci * _SEG_ZCH, _SEG_ZCH)
            pltpu.sync_copy(z_ref.at[sl], acc_sh.at[sl])

    plsc.subcore_barrier()

    def chunk_loop(idx3_hbm):
        def load_idx(phase):
            pltpu.sync_copy(idx3_hbm.at[2 * t + phase], idx_all)

        def load(k, buf, lsem):
            base = t * _SEG_EPT + k * _SEG_CB
            return pltpu.async_copy(e_ref.at[pl.ds(base, _SEG_CB)], buf, lsem)

        def scat_start(k, buf, ssem):
            pltpu.async_copy(
                buf, acc_sh.at[idx_all.at[lax.rem(k, _SEG_IPH)]],
                ssem, add=True)

        def scat_wait(k, buf, ssem):
            pltpu.make_async_copy(
                buf, acc_sh.at[idx_all.at[lax.rem(k, _SEG_IPH)]],
                ssem).wait()

        def half(k, buf, lsem, ssem, obuf, olsem, ossem):
            base = t * _SEG_EPT + k * _SEG_CB
            pltpu.make_async_copy(
                e_ref.at[pl.ds(base, _SEG_CB)], buf, lsem).wait()

            @pl.when(k >= 1)
            def _():
                scat_wait(k - 1, obuf, ossem)

            @pl.when(k == _SEG_IPH)
            def _():
                load_idx(1)

            scat_start(k, buf, ssem)

            @pl.when(k + 1 < _SEG_NST)
            def _():
                load(k + 1, obuf, olsem)

        load_idx(0)
        load(0, rows0, lsem0)

        def dstep(i, _):
            half(2 * i, rows0, lsem0, ssem0, rows1, lsem1, ssem1)
            half(2 * i + 1, rows1, lsem1, ssem1, rows0, lsem0, ssem0)
            return 0

        lax.fori_loop(0, _SEG_NST // 2, dstep, 0)
        scat_wait(_SEG_NST - 1, rows1, ssem1)

    @pl.when(c == 0)
    def _():
        chunk_loop(s2_ref)

    @pl.when(c == 1)
    def _():
        chunk_loop(r2_ref)

    plsc.subcore_barrier()

    for k in range(2):
        ci = t + k * NS

        @pl.when(ci < _SEG_NCH)
        def _():
            sl = pl.ds(ci * _SEG_ZCH, _SEG_ZCH)

            @pl.when(c == 0)
            def _():
                pltpu.sync_copy(acc_sh.at[sl], so_ref.at[sl])

            @pl.when(c == 1)
            def _():
                pltpu.sync_copy(acc_sh.at[sl], ro_ref.at[sl])


def _segsum(edges_t, senders2d, receivers2d, zeros_nd):
    f = pl.kernel(
        _segsum_body,
        out_type=(jax.ShapeDtypeStruct((N_NODES, D), jnp.float32),
                  jax.ShapeDtypeStruct((N_NODES, D), jnp.float32)),
        mesh=_MESH,
        scratch_types=[
            pltpu.VMEM((_SEG_IPH, _SEG_CB), jnp.int32),
            pltpu.VMEM((_SEG_CB, D), jnp.float32),
            pltpu.VMEM((_SEG_CB, D), jnp.float32),
            pltpu.VMEM_SHARED((N_NODES, D), jnp.float32),
            pltpu.SemaphoreType.DMA,
            pltpu.SemaphoreType.DMA,
            pltpu.SemaphoreType.DMA,
            pltpu.SemaphoreType.DMA,
        ],
    )
    return f(edges_t, senders2d, receivers2d, zeros_nd)


_G_CH = 200
_G_GB = 40
_G_EPW = N_EDGES // NW
_G_NST = _G_EPW // _G_CH


def _gather_body(a_ref, b_ref, s_ref, r_ref, ga_ref, gb_ref,
                 sidx_v, ridx_v, bufa0, bufb0, bufa1, bufb1,
                 gsem0, gsem1, wsem0, wsem1):
    c = lax.axis_index("c")
    t = lax.axis_index("s")
    wid = t * NC + c
    tbase = wid * _G_EPW

    pltpu.sync_copy(s_ref.at[pl.ds(tbase, _G_EPW)], sidx_v)
    pltpu.sync_copy(r_ref.at[pl.ds(tbase, _G_EPW)], ridx_v)

    nb = _G_CH // _G_GB

    def g_descs(k, bufa, bufb, gsem):
        ds = []
        for j in range(nb):
            off = k * _G_CH + j * _G_GB
            bsl = pl.ds(j * _G_GB, _G_GB)
            ds.append(pltpu.make_async_copy(
                a_ref.at[sidx_v.at[pl.ds(off, _G_GB)]], bufa.at[bsl], gsem))
            ds.append(pltpu.make_async_copy(
                b_ref.at[ridx_v.at[pl.ds(off, _G_GB)]], bufb.at[bsl], gsem))
        return ds

    def w_descs(k, bufa, bufb, wsem):
        sl = pl.ds(tbase + k * _G_CH, _G_CH)
        return [pltpu.make_async_copy(bufa, ga_ref.at[sl], wsem),
                pltpu.make_async_copy(bufb, gb_ref.at[sl], wsem)]

    bufs = ((bufa0, bufb0, gsem0, wsem0), (bufa1, bufb1, gsem1, wsem1))

    def step(k, cur, oth):
        ca, cb, cg, cw = cur
        oa, ob, og, ow = oth
        for d in g_descs(k, ca, cb, cg):
            d.wait()
        for d in w_descs(k, ca, cb, cw):
            d.start()

        @pl.when(k > 0)
        def _():
            for d in w_descs(k - 1, oa, ob, ow):
                d.wait()

        @pl.when(k + 1 < _G_NST)
        def _():
            for d in g_descs(k + 1, oa, ob, og):
                d.start()

    for d in g_descs(0, bufa0, bufb0, gsem0):
        d.start()

    def dstep(i, _):
        step(2 * i, bufs[0], bufs[1])
        step(2 * i + 1, bufs[1], bufs[0])
        return 0

    lax.fori_loop(0, _G_NST // 2, dstep, 0)

    for d in w_descs(_G_NST - 1, bufa1, bufb1, wsem1):
        d.wait()


def _gather(a_t, b_t, senders, receivers):
    f = pl.kernel(
        _gather_body,
        out_type=(jax.ShapeDtypeStruct((N_EDGES, D), jnp.float32),
                  jax.ShapeDtypeStruct((N_EDGES, D), jnp.float32)),
        mesh=_MESH,
        scratch_types=[
            pltpu.VMEM((_G_EPW,), jnp.int32),
            pltpu.VMEM((_G_EPW,), jnp.int32),
            pltpu.VMEM((_G_CH, D), jnp.float32),
            pltpu.VMEM((_G_CH, D), jnp.float32),
            pltpu.VMEM((_G_CH, D), jnp.float32),
            pltpu.VMEM((_G_CH, D), jnp.float32),
            pltpu.SemaphoreType.DMA,
            pltpu.SemaphoreType.DMA,
            pltpu.SemaphoreType.DMA,
            pltpu.SemaphoreType.DMA,
        ],
    )
    return f(a_t, b_t, senders, receivers)


def kernel(nodes, edges, receivers, senders, W_node, W_edge):
    wn_t = W_node.T
    we_t = W_edge.T
    we0_t = we_t[:D]
    zeros_nd = jnp.zeros((N_NODES, D), jnp.float32)
    senders2d = senders.reshape(NS * 2, _SEG_IPH, _SEG_CB)
    receivers2d = receivers.reshape(NS * 2, _SEG_IPH, _SEG_CB)

    edges_t = _transpose(edges, 2560)
    nodes_t = _transpose(nodes, N_NODES)

    nodes_out = None
    edges_out = None
    for rnd in range(2):
        last = rnd == 1
        s_t, r_t = _segsum(edges_t, senders2d, receivers2d, zeros_nd)
        n_out, a_t, b_t = _node_update(nodes_t, s_t, r_t, wn_t, we_t, last)
        ga, gb = _gather(a_t, b_t, senders, receivers)
        e_out = _edge_update(edges_t, ga, gb, we0_t, last)
        if last:
            nodes_out, edges_out = n_out, e_out
        else:
            nodes_t, edges_t = n_out, e_out

    return nodes_out, edges_out, receivers, senders


# speedup vs baseline: 4.0936x; 1.1242x over previous
import functools

import jax
import jax.numpy as jnp
from jax import lax
from jax.experimental import pallas as pl
from jax.experimental.pallas import tpu as pltpu
from jax.experimental.pallas import tpu_sc as plsc

N_NODES = 10000
N_EDGES = 320000
D = 128

NC = 2
NS = 16
NW = NC * NS


def _transpose_body(x_ref, o_ref):
    o_ref[...] = x_ref[...].T


def _transpose(x, bn):
    d, m = x.shape
    return pl.pallas_call(
        _transpose_body,
        grid=(m // bn,),
        in_specs=[pl.BlockSpec((d, bn), lambda i: (0, i))],
        out_specs=pl.BlockSpec((bn, d), lambda i: (i, 0)),
        out_shape=jax.ShapeDtypeStruct((m, d), x.dtype),
    )(x)


def _node_update_body(last, chain, refs):
    if chain:
        (n_ref, s_ref, ps_ref, r_ref, pr_ref, wn_ref, we_ref,
         no_ref, a_ref, b_ref) = refs
    else:
        n_ref, s_ref, r_ref, wn_ref, we_ref, no_ref, a_ref, b_ref = refs
    wn = wn_ref[...]
    we = we_ref[...]
    s = s_ref[...]
    r = r_ref[...]
    if chain:
        we0 = we[:D]
        s = jnp.dot(s, we0, preferred_element_type=jnp.float32) + ps_ref[...]
        r = jnp.dot(r, we0, preferred_element_type=jnp.float32) + pr_ref[...]
    x = jnp.dot(n_ref[...], wn[:D], preferred_element_type=jnp.float32)
    x = x + jnp.dot(s, wn[D:2 * D], preferred_element_type=jnp.float32)
    x = x + jnp.dot(r, wn[2 * D:], preferred_element_type=jnp.float32)
    if last:
        no_ref[...] = x.T
    else:
        no_ref[...] = x
    a_ref[...] = jnp.dot(x, we[D:2 * D], preferred_element_type=jnp.float32)
    b_ref[...] = jnp.dot(x, we[2 * D:], preferred_element_type=jnp.float32)


def _node_update(nodes_t, s_t, r_t, wn_t, we_t, last, ps_t=None, pr_t=None):
    bn = N_NODES
    grid = (N_NODES // bn,)
    chain = ps_t is not None
    row_spec = pl.BlockSpec((bn, D), lambda i: (i, 0))
    w_spec = pl.BlockSpec((3 * D, D), lambda i: (0, 0))
    if last:
        no_spec = pl.BlockSpec((D, bn), lambda i: (0, i))
        no_shape = jax.ShapeDtypeStruct((D, N_NODES), jnp.float32)
    else:
        no_spec = row_spec
        no_shape = jax.ShapeDtypeStruct((N_NODES, D), jnp.float32)
    if chain:
        args = (nodes_t, s_t, ps_t, r_t, pr_t, wn_t, we_t)
        in_specs = [row_spec] * 5 + [w_spec, w_spec]
    else:
        args = (nodes_t, s_t, r_t, wn_t, we_t)
        in_specs = [row_spec] * 3 + [w_spec, w_spec]
    return pl.pallas_call(
        lambda *refs: _node_update_body(last, chain, refs),
        grid=grid,
        in_specs=in_specs,
        out_specs=(no_spec, row_spec, row_spec),
        out_shape=(no_shape,
                   jax.ShapeDtypeStruct((N_NODES, D), jnp.float32),
                   jax.ShapeDtypeStruct((N_NODES, D), jnp.float32)),
    )(*args)


def _edge_update_body(last, e_ref, g_ref, w_ref, o_ref):
    y = jnp.dot(e_ref[...], w_ref[...], preferred_element_type=jnp.float32)
    y = y + g_ref[...]
    if last:
        o_ref[...] = y.T
    else:
        o_ref[...] = y


def _edge_update(edges_t, g, we0_t, last):
    be = 2560
    grid = (N_EDGES // be,)
    row_spec = pl.BlockSpec((be, D), lambda i: (i, 0))
    w_spec = pl.BlockSpec((D, D), lambda i: (0, 0))
    if last:
        o_spec = pl.BlockSpec((D, be), lambda i: (0, i))
        o_shape = jax.ShapeDtypeStruct((D, N_EDGES), jnp.float32)
    else:
        o_spec = row_spec
        o_shape = jax.ShapeDtypeStruct((N_EDGES, D), jnp.float32)
    return pl.pallas_call(
        functools.partial(_edge_update_body, last),
        grid=grid,
        in_specs=[row_spec, row_spec, w_spec],
        out_specs=o_spec,
        out_shape=o_shape,
    )(edges_t, g, we0_t)


_SEG_CB = 80
_SEG_EPT = N_EDGES // NS
_SEG_NST = _SEG_EPT // _SEG_CB
_SEG_IPH = _SEG_NST // 2
_SEG_ZCH = 400
_SEG_NCH = N_NODES // _SEG_ZCH

_MESH = plsc.VectorSubcoreMesh(
    core_axis_name="c", subcore_axis_name="s", num_cores=NC)


def _segsum_body(e_ref, s2_ref, r2_ref, z_ref, so_ref, ro_ref,
                 idx_all, rows0, rows1, acc_sh, lsem0, lsem1, ssem0, ssem1):
    c = lax.axis_index("c")
    t = lax.axis_index("s")

    for k in range(2):
        ci = t + k * NS

        @pl.when(ci < _SEG_NCH)
        def _():
            sl = pl.ds(ci * _SEG_ZCH, _SEG_ZCH)
            pltpu.sync_copy(z_ref.at[sl], acc_sh.at[sl])

    plsc.subcore_barrier()

    def chunk_loop(idx3_hbm):
        def load_idx(phase):
            pltpu.sync_copy(idx3_hbm.at[2 * t + phase], idx_all)

        def load(k, buf, lsem):
            base = t * _SEG_EPT + k * _SEG_CB
            return pltpu.async_copy(e_ref.at[pl.ds(base, _SEG_CB)], buf, lsem)

        def scat_start(k, buf, ssem):
            pltpu.async_copy(
                buf, acc_sh.at[idx_all.at[lax.rem(k, _SEG_IPH)]],
                ssem, add=True)

        def scat_wait(k, buf, ssem):
            pltpu.make_async_copy(
                buf, acc_sh.at[idx_all.at[lax.rem(k, _SEG_IPH)]],
                ssem).wait()

        def half(k, buf, lsem, ssem, obuf, olsem, ossem):
            base = t * _SEG_EPT + k * _SEG_CB
            pltpu.make_async_copy(
                e_ref.at[pl.ds(base, _SEG_CB)], buf, lsem).wait()

            @pl.when(k >= 1)
            def _():
                scat_wait(k - 1, obuf, ossem)

            @pl.when(k == _SEG_IPH)
            def _():
                load_idx(1)

            scat_start(k, buf, ssem)

            @pl.when(k + 1 < _SEG_NST)
            def _():
                load(k + 1, obuf, olsem)

        load_idx(0)
        load(0, rows0, lsem0)

        def dstep(i, _):
            half(2 * i, rows0, lsem0, ssem0, rows1, lsem1, ssem1)
            half(2 * i + 1, rows1, lsem1, ssem1, rows0, lsem0, ssem0)
            return 0

        lax.fori_loop(0, _SEG_NST // 2, dstep, 0)
        scat_wait(_SEG_NST - 1, rows1, ssem1)

    @pl.when(c == 0)
    def _():
        chunk_loop(s2_ref)

    @pl.when(c == 1)
    def _():
        chunk_loop(r2_ref)

    plsc.subcore_barrier()

    for k in range(2):
        ci = t + k * NS

        @pl.when(ci < _SEG_NCH)
        def _():
            sl = pl.ds(ci * _SEG_ZCH, _SEG_ZCH)

            @pl.when(c == 0)
            def _():
                pltpu.sync_copy(acc_sh.at[sl], so_ref.at[sl])

            @pl.when(c == 1)
            def _():
                pltpu.sync_copy(acc_sh.at[sl], ro_ref.at[sl])


def _segsum(edges_t, senders2d, receivers2d, zeros_nd):
    f = pl.kernel(
        _segsum_body,
        out_type=(jax.ShapeDtypeStruct((N_NODES, D), jnp.float32),
                  jax.ShapeDtypeStruct((N_NODES, D), jnp.float32)),
        mesh=_MESH,
        scratch_types=[
            pltpu.VMEM((_SEG_IPH, _SEG_CB), jnp.int32),
            pltpu.VMEM((_SEG_CB, D), jnp.float32),
            pltpu.VMEM((_SEG_CB, D), jnp.float32),
            pltpu.VMEM_SHARED((N_NODES, D), jnp.float32),
            pltpu.SemaphoreType.DMA,
            pltpu.SemaphoreType.DMA,
            pltpu.SemaphoreType.DMA,
            pltpu.SemaphoreType.DMA,
        ],
    )
    return f(edges_t, senders2d, receivers2d, zeros_nd)


_G_CH = 400
_G_GB = 40
_G_EPW = N_EDGES // NW
_G_NST = _G_EPW // _G_CH


def _gather_body(a_ref, b_ref, s_ref, r_ref, g_ref,
                 sidx_v, ridx_v, buf0, buf1,
                 gsem0, gsem1, asem0, asem1, wsem0, wsem1):
    c = lax.axis_index("c")
    t = lax.axis_index("s")
    wid = t * NC + c
    tbase = wid * _G_EPW

    pltpu.sync_copy(s_ref.at[pl.ds(tbase, _G_EPW)], sidx_v)
    pltpu.sync_copy(r_ref.at[pl.ds(tbase, _G_EPW)], ridx_v)

    nb = _G_CH // _G_GB

    def a_descs(k, buf, gsem):
        ds = []
        for j in range(nb):
            off = k * _G_CH + j * _G_GB
            bsl = pl.ds(j * _G_GB, _G_GB)
            ds.append(pltpu.make_async_copy(
                a_ref.at[sidx_v.at[pl.ds(off, _G_GB)]], buf.at[bsl], gsem))
        return ds

    def badd_start(k, buf, asem):
        for j in range(nb):
            off = k * _G_CH + j * _G_GB
            bsl = pl.ds(j * _G_GB, _G_GB)
            pltpu.async_copy(
                b_ref.at[ridx_v.at[pl.ds(off, _G_GB)]], buf.at[bsl],
                asem, add=True)

    def badd_wait(k, buf, asem):
        for j in range(nb):
            off = k * _G_CH + j * _G_GB
            bsl = pl.ds(j * _G_GB, _G_GB)
            pltpu.make_async_copy(
                b_ref.at[ridx_v.at[pl.ds(off, _G_GB)]], buf.at[bsl],
                asem).wait()

    def w_desc(k, buf, wsem):
        sl = pl.ds(tbase + k * _G_CH, _G_CH)
        return pltpu.make_async_copy(buf, g_ref.at[sl], wsem)

    bufs = ((buf0, gsem0, asem0, wsem0), (buf1, gsem1, asem1, wsem1))

    def step(k, cur, oth):
        cb, cg, ca, cw = cur
        ob, og, oa, ow = oth
        for d in a_descs(k, cb, cg):
            d.wait()
        badd_start(k, cb, ca)
        badd_wait(k, cb, ca)
        w_desc(k, cb, cw).start()

        @pl.when(k > 0)
        def _():
            w_desc(k - 1, ob, ow).wait()

        @pl.when(k + 1 < _G_NST)
        def _():
            for d in a_descs(k + 1, ob, og):
                d.start()

    for d in a_descs(0, buf0, gsem0):
        d.start()

    def dstep(i, _):
        step(2 * i, bufs[0], bufs[1])
        step(2 * i + 1, bufs[1], bufs[0])
        return 0

    lax.fori_loop(0, _G_NST // 2, dstep, 0)
    step(_G_NST - 1, bufs[(_G_NST - 1) % 2], bufs[_G_NST % 2])
    w_desc(_G_NST - 1, bufs[(_G_NST - 1) % 2][0],
           bufs[(_G_NST - 1) % 2][3]).wait()


def _gather(a_t, b_t, senders, receivers):
    f = pl.kernel(
        _gather_body,
        out_type=jax.ShapeDtypeStruct((N_EDGES, D), jnp.float32),
        mesh=_MESH,
        scratch_types=[
            pltpu.VMEM((_G_EPW,), jnp.int32),
            pltpu.VMEM((_G_EPW,), jnp.int32),
            pltpu.VMEM((_G_CH, D), jnp.float32),
            pltpu.VMEM((_G_CH, D), jnp.float32),
            pltpu.SemaphoreType.DMA,
            pltpu.SemaphoreType.DMA,
            pltpu.SemaphoreType.DMA,
            pltpu.SemaphoreType.DMA,
            pltpu.SemaphoreType.DMA,
            pltpu.SemaphoreType.DMA,
        ],
    )
    return f(a_t, b_t, senders, receivers)


def kernel(nodes, edges, receivers, senders, W_node, W_edge):
    wn_t = W_node.T
    we_t = W_edge.T
    we0_t = we_t[:D]
    zeros_nd = jnp.zeros((N_NODES, D), jnp.float32)
    senders2d = senders.reshape(NS * 2, _SEG_IPH, _SEG_CB)
    receivers2d = receivers.reshape(NS * 2, _SEG_IPH, _SEG_CB)

    edges_t = _transpose(edges, 2560)
    nodes_t = _transpose(nodes, N_NODES)

    s1, r1 = _segsum(edges_t, senders2d, receivers2d, zeros_nd)
    n1_t, a1, b1 = _node_update(nodes_t, s1, r1, wn_t, we_t, False)
    g1 = _gather(a1, b1, senders, receivers)
    ps, pr = _segsum(g1, senders2d, receivers2d, zeros_nd)
    edges_t1 = _edge_update(edges_t, g1, we0_t, False)
    nodes_out, a2, b2 = _node_update(n1_t, s1, r1, wn_t, we_t, True, ps, pr)
    g2 = _gather(a2, b2, senders, receivers)
    edges_out = _edge_update(edges_t1, g2, we0_t, True)

    return nodes_out, edges_out, receivers, senders


# speedup vs baseline: 4.1225x; 1.0071x over previous
import functools

import jax
import jax.numpy as jnp
from jax import lax
from jax.experimental import pallas as pl
from jax.experimental.pallas import tpu as pltpu
from jax.experimental.pallas import tpu_sc as plsc

N_NODES = 10000
N_EDGES = 320000
D = 128

NC = 2
NS = 16
NW = NC * NS


def _transpose_body(x_ref, o_ref):
    o_ref[...] = x_ref[...].T


def _transpose(x, bn):
    d, m = x.shape
    return pl.pallas_call(
        _transpose_body,
        grid=(m // bn,),
        in_specs=[pl.BlockSpec((d, bn), lambda i: (0, i))],
        out_specs=pl.BlockSpec((bn, d), lambda i: (i, 0)),
        out_shape=jax.ShapeDtypeStruct((m, d), x.dtype),
    )(x)


def _node_update_body(last, chain, refs):
    if chain:
        (n_ref, s_ref, ps_ref, r_ref, pr_ref, wn_ref, we_ref,
         no_ref, a_ref, b_ref) = refs
    else:
        n_ref, s_ref, r_ref, wn_ref, we_ref, no_ref, a_ref, b_ref = refs
    wn = wn_ref[...]
    we = we_ref[...]
    s = s_ref[...]
    r = r_ref[...]
    if chain:
        we0 = we[:D]
        s = jnp.dot(s, we0, preferred_element_type=jnp.float32) + ps_ref[...]
        r = jnp.dot(r, we0, preferred_element_type=jnp.float32) + pr_ref[...]
    x = jnp.dot(n_ref[...], wn[:D], preferred_element_type=jnp.float32)
    x = x + jnp.dot(s, wn[D:2 * D], preferred_element_type=jnp.float32)
    x = x + jnp.dot(r, wn[2 * D:], preferred_element_type=jnp.float32)
    if last:
        no_ref[...] = x.T
    else:
        no_ref[...] = x
    a_ref[...] = jnp.dot(x, we[D:2 * D], preferred_element_type=jnp.float32)
    b_ref[...] = jnp.dot(x, we[2 * D:], preferred_element_type=jnp.float32)


def _node_update(nodes_t, s_t, r_t, wn_t, we_t, last, ps_t=None, pr_t=None):
    bn = N_NODES
    grid = (N_NODES // bn,)
    chain = ps_t is not None
    row_spec = pl.BlockSpec((bn, D), lambda i: (i, 0))
    w_spec = pl.BlockSpec((3 * D, D), lambda i: (0, 0))
    if last:
        no_spec = pl.BlockSpec((D, bn), lambda i: (0, i))
        no_shape = jax.ShapeDtypeStruct((D, N_NODES), jnp.float32)
    else:
        no_spec = row_spec
        no_shape = jax.ShapeDtypeStruct((N_NODES, D), jnp.float32)
    if chain:
        args = (nodes_t, s_t, ps_t, r_t, pr_t, wn_t, we_t)
        in_specs = [row_spec] * 5 + [w_spec, w_spec]
    else:
        args = (nodes_t, s_t, r_t, wn_t, we_t)
        in_specs = [row_spec] * 3 + [w_spec, w_spec]
    return pl.pallas_call(
        lambda *refs: _node_update_body(last, chain, refs),
        grid=grid,
        in_specs=in_specs,
        out_specs=(no_spec, row_spec, row_spec),
        out_shape=(no_shape,
                   jax.ShapeDtypeStruct((N_NODES, D), jnp.float32),
                   jax.ShapeDtypeStruct((N_NODES, D), jnp.float32)),
    )(*args)


def _edge_mid_body(e_ref, g_ref, w_ref, o_ref):
    y = jnp.dot(e_ref[...], w_ref[...], preferred_element_type=jnp.float32)
    y = y + g_ref[...]
    o_ref[...] = jnp.dot(y, w_ref[...], preferred_element_type=jnp.float32)


def _edge_mid(edges_t, g, we0_t):
    be = 2560
    grid = (N_EDGES // be,)
    row_spec = pl.BlockSpec((be, D), lambda i: (i, 0))
    w_spec = pl.BlockSpec((D, D), lambda i: (0, 0))
    return pl.pallas_call(
        _edge_mid_body,
        grid=grid,
        in_specs=[row_spec, row_spec, w_spec],
        out_specs=row_spec,
        out_shape=jax.ShapeDtypeStruct((N_EDGES, D), jnp.float32),
    )(edges_t, g, we0_t)


def _edge_final_body(m_ref, g_ref, o_ref):
    o_ref[...] = (m_ref[...] + g_ref[...]).T


def _edge_final(m2, g2):
    be = 2560
    grid = (N_EDGES // be,)
    row_spec = pl.BlockSpec((be, D), lambda i: (i, 0))
    return pl.pallas_call(
        _edge_final_body,
        grid=grid,
        in_specs=[row_spec, row_spec],
        out_specs=pl.BlockSpec((D, be), lambda i: (0, i)),
        out_shape=jax.ShapeDtypeStruct((D, N_EDGES), jnp.float32),
    )(m2, g2)


_SEG_CB = 80
_SEG_EPT = N_EDGES // NS
_SEG_NST = _SEG_EPT // _SEG_CB
_SEG_IPH = _SEG_NST // 2
_SEG_ZCH = 400
_SEG_NCH = N_NODES // _SEG_ZCH

_MESH = plsc.VectorSubcoreMesh(
    core_axis_name="c", subcore_axis_name="s", num_cores=NC)


def _segsum_body(e_ref, s2_ref, r2_ref, z_ref, so_ref, ro_ref,
                 idx_all, rows0, rows1, acc_sh, lsem0, lsem1, ssem0, ssem1):
    c = lax.axis_index("c")
    t = lax.axis_index("s")

    for k in range(2):
        ci = t + k * NS

        @pl.when(ci < _SEG_NCH)
        def _():
            sl = pl.ds(ci * _SEG_ZCH, _SEG_ZCH)
            pltpu.sync_copy(z_ref.at[sl], acc_sh.at[sl])

    plsc.subcore_barrier()

    def chunk_loop(idx3_hbm):
        def load_idx(phase):
            pltpu.sync_copy(idx3_hbm.at[2 * t + phase], idx_all)

        def load(k, buf, lsem):
            base = t * _SEG_EPT + k * _SEG_CB
            return pltpu.async_copy(e_ref.at[pl.ds(base, _SEG_CB)], buf, lsem)

        def scat_start(k, buf, ssem):
            pltpu.async_copy(
                buf, acc_sh.at[idx_all.at[lax.rem(k, _SEG_IPH)]],
                ssem, add=True)

        def scat_wait(k, buf, ssem):
            pltpu.make_async_copy(
                buf, acc_sh.at[idx_all.at[lax.rem(k, _SEG_IPH)]],
                ssem).wait()

        def half(k, buf, lsem, ssem, obuf, olsem, ossem):
            base = t * _SEG_EPT + k * _SEG_CB
            pltpu.make_async_copy(
                e_ref.at[pl.ds(base, _SEG_CB)], buf, lsem).wait()

            @pl.when(k >= 1)
            def _():
                scat_wait(k - 1, obuf, ossem)

            @pl.when(k == _SEG_IPH)
            def _():
                load_idx(1)

            scat_start(k, buf, ssem)

            @pl.when(k + 1 < _SEG_NST)
            def _():
                load(k + 1, obuf, olsem)

        load_idx(0)
        load(0, rows0, lsem0)

        def dstep(i, _):
            half(2 * i, rows0, lsem0, ssem0, rows1, lsem1, ssem1)
            half(2 * i + 1, rows1, lsem1, ssem1, rows0, lsem0, ssem0)
            return 0

        lax.fori_loop(0, _SEG_NST // 2, dstep, 0)
        scat_wait(_SEG_NST - 1, rows1, ssem1)

    @pl.when(c == 0)
    def _():
        chunk_loop(s2_ref)

    @pl.when(c == 1)
    def _():
        chunk_loop(r2_ref)

    plsc.subcore_barrier()

    for k in range(2):
        ci = t + k * NS

        @pl.when(ci < _SEG_NCH)
        def _():
            sl = pl.ds(ci * _SEG_ZCH, _SEG_ZCH)

            @pl.when(c == 0)
            def _():
                pltpu.sync_copy(acc_sh.at[sl], so_ref.at[sl])

            @pl.when(c == 1)
            def _():
                pltpu.sync_copy(acc_sh.at[sl], ro_ref.at[sl])


def _segsum(edges_t, senders2d, receivers2d, zeros_nd):
    f = pl.kernel(
        _segsum_body,
        out_type=(jax.ShapeDtypeStruct((N_NODES, D), jnp.float32),
                  jax.ShapeDtypeStruct((N_NODES, D), jnp.float32)),
        mesh=_MESH,
        scratch_types=[
            pltpu.VMEM((_SEG_IPH, _SEG_CB), jnp.int32),
            pltpu.VMEM((_SEG_CB, D), jnp.float32),
            pltpu.VMEM((_SEG_CB, D), jnp.float32),
            pltpu.VMEM_SHARED((N_NODES, D), jnp.float32),
            pltpu.SemaphoreType.DMA,
            pltpu.SemaphoreType.DMA,
            pltpu.SemaphoreType.DMA,
            pltpu.SemaphoreType.DMA,
        ],
    )
    return f(edges_t, senders2d, receivers2d, zeros_nd)


_G_CH = 400
_G_GB = 40
_G_EPW = N_EDGES // NW
_G_NST = _G_EPW // _G_CH


def _gather_body(a_ref, b_ref, s_ref, r_ref, g_ref,
                 sidx_v, ridx_v, buf0, buf1,
                 gsem0, gsem1, asem0, asem1, wsem0, wsem1):
    c = lax.axis_index("c")
    t = lax.axis_index("s")
    wid = t * NC + c
    tbase = wid * _G_EPW

    pltpu.sync_copy(s_ref.at[pl.ds(tbase, _G_EPW)], sidx_v)
    pltpu.sync_copy(r_ref.at[pl.ds(tbase, _G_EPW)], ridx_v)

    nb = _G_CH // _G_GB

    def a_descs(k, buf, gsem):
        ds = []
        for j in range(nb):
            off = k * _G_CH + j * _G_GB
            bsl = pl.ds(j * _G_GB, _G_GB)
            ds.append(pltpu.make_async_copy(
                a_ref.at[sidx_v.at[pl.ds(off, _G_GB)]], buf.at[bsl], gsem))
        return ds

    def badd_start(k, buf, asem):
        for j in range(nb):
            off = k * _G_CH + j * _G_GB
            bsl = pl.ds(j * _G_GB, _G_GB)
            pltpu.async_copy(
                b_ref.at[ridx_v.at[pl.ds(off, _G_GB)]], buf.at[bsl],
                asem, add=True)

    def badd_wait(k, buf, asem):
        for j in range(nb):
            off = k * _G_CH + j * _G_GB
            bsl = pl.ds(j * _G_GB, _G_GB)
            pltpu.make_async_copy(
                b_ref.at[ridx_v.at[pl.ds(off, _G_GB)]], buf.at[bsl],
                asem).wait()

    def w_desc(k, buf, wsem):
        sl = pl.ds(tbase + k * _G_CH, _G_CH)
        return pltpu.make_async_copy(buf, g_ref.at[sl], wsem)

    bufs = ((buf0, gsem0, asem0, wsem0), (buf1, gsem1, asem1, wsem1))

    def step(k, cur, oth):
        cb, cg, ca, cw = cur
        ob, og, oa, ow = oth
        for d in a_descs(k, cb, cg):
            d.wait()
        badd_start(k, cb, ca)
        badd_wait(k, cb, ca)
        w_desc(k, cb, cw).start()

        @pl.when(k > 0)
        def _():
            w_desc(k - 1, ob, ow).wait()

        @pl.when(k + 1 < _G_NST)
        def _():
            for d in a_descs(k + 1, ob, og):
                d.start()

    for d in a_descs(0, buf0, gsem0):
        d.start()

    def dstep(i, _):
        step(2 * i, bufs[0], bufs[1])
        step(2 * i + 1, bufs[1], bufs[0])
        return 0

    lax.fori_loop(0, _G_NST // 2, dstep, 0)
    step(_G_NST - 1, bufs[(_G_NST - 1) % 2], bufs[_G_NST % 2])
    w_desc(_G_NST - 1, bufs[(_G_NST - 1) % 2][0],
           bufs[(_G_NST - 1) % 2][3]).wait()


def _gather(a_t, b_t, senders, receivers):
    f = pl.kernel(
        _gather_body,
        out_type=jax.ShapeDtypeStruct((N_EDGES, D), jnp.float32),
        mesh=_MESH,
        scratch_types=[
            pltpu.VMEM((_G_EPW,), jnp.int32),
            pltpu.VMEM((_G_EPW,), jnp.int32),
            pltpu.VMEM((_G_CH, D), jnp.float32),
            pltpu.VMEM((_G_CH, D), jnp.float32),
            pltpu.SemaphoreType.DMA,
            pltpu.SemaphoreType.DMA,
            pltpu.SemaphoreType.DMA,
            pltpu.SemaphoreType.DMA,
            pltpu.SemaphoreType.DMA,
            pltpu.SemaphoreType.DMA,
        ],
    )
    return f(a_t, b_t, senders, receivers)


def kernel(nodes, edges, receivers, senders, W_node, W_edge):
    wn_t = W_node.T
    we_t = W_edge.T
    we0_t = we_t[:D]
    zeros_nd = jnp.zeros((N_NODES, D), jnp.float32)
    senders2d = senders.reshape(NS * 2, _SEG_IPH, _SEG_CB)
    receivers2d = receivers.reshape(NS * 2, _SEG_IPH, _SEG_CB)

    edges_t = _transpose(edges, 2560)
    nodes_t = _transpose(nodes, N_NODES)

    s1, r1 = _segsum(edges_t, senders2d, receivers2d, zeros_nd)
    n1_t, a1, b1 = _node_update(nodes_t, s1, r1, wn_t, we_t, False)
    g1 = _gather(a1, b1, senders, receivers)
    ps, pr = _segsum(g1, senders2d, receivers2d, zeros_nd)
    m2 = _edge_mid(edges_t, g1, we0_t)
    nodes_out, a2, b2 = _node_update(n1_t, s1, r1, wn_t, we_t, True, ps, pr)
    g2 = _gather(a2, b2, senders, receivers)
    edges_out = _edge_final(m2, g2)

    return nodes_out, edges_out, receivers, senders
